# baseline (speedup 1.0000x reference)
import functools

import jax
import jax.numpy as jnp
from jax import lax
from jax.experimental import pallas as pl
from jax.experimental.pallas import tpu as pltpu, tpu_sc as plsc

NUM_FIELDS = 26
EMB = 32
BATCH = 16384

_info = plsc.get_sparse_core_info()
_NC, _NS = _info.num_cores, _info.num_subcores
_NW = _NC * _NS
_BPW = BATCH // _NW

_mesh = plsc.VectorSubcoreMesh(core_axis_name="c", subcore_axis_name="s")


@functools.partial(
    pl.kernel,
    mesh=_mesh,
    out_type=jax.ShapeDtypeStruct((BATCH, NUM_FIELDS * EMB), jnp.float32),
    compiler_params=pltpu.CompilerParams(use_tc_tiling_on_sc=False),
    scratch_types=[
        pltpu.VMEM((NUM_FIELDS, _BPW), jnp.int32),
        pltpu.VMEM((_BPW, EMB), jnp.float32),
        pltpu.SemaphoreType.DMA,
    ],
)
def _gather_concat(idx_hbm, *rest):
    tables = rest[:NUM_FIELDS]
    out_hbm = rest[NUM_FIELDS]
    idx_v, rows_v, sem = rest[NUM_FIELDS + 1:]

    wid = lax.axis_index("s") * _NC + lax.axis_index("c")
    base = wid * _BPW

    pltpu.sync_copy(idx_hbm.at[:, pl.ds(base, _BPW)], idx_v)

    for f in range(NUM_FIELDS):
        pltpu.async_copy(tables[f].at[idx_v.at[f]], rows_v, sem).wait()
        pltpu.sync_copy(
            rows_v, out_hbm.at[pl.ds(base, _BPW), pl.ds(f * EMB, EMB)]
        )


def kernel(
    field_00, field_01, field_02, field_03, field_04, field_05, field_06,
    field_07, field_08, field_09, field_10, field_11, field_12, field_13,
    field_14, field_15, field_16, field_17, field_18, field_19, field_20,
    field_21, field_22, field_23, field_24, field_25,
    W_field_00, W_field_01, W_field_02, W_field_03, W_field_04, W_field_05,
    W_field_06, W_field_07, W_field_08, W_field_09, W_field_10, W_field_11,
    W_field_12, W_field_13, W_field_14, W_field_15, W_field_16, W_field_17,
    W_field_18, W_field_19, W_field_20, W_field_21, W_field_22, W_field_23,
    W_field_24, W_field_25,
):
    indices = jnp.stack([
        field_00, field_01, field_02, field_03, field_04, field_05, field_06,
        field_07, field_08, field_09, field_10, field_11, field_12, field_13,
        field_14, field_15, field_16, field_17, field_18, field_19, field_20,
        field_21, field_22, field_23, field_24, field_25,
    ])
    tables = (
        W_field_00, W_field_01, W_field_02, W_field_03, W_field_04, W_field_05,
        W_field_06, W_field_07, W_field_08, W_field_09, W_field_10, W_field_11,
        W_field_12, W_field_13, W_field_14, W_field_15, W_field_16, W_field_17,
        W_field_18, W_field_19, W_field_20, W_field_21, W_field_22, W_field_23,
        W_field_24, W_field_25,
    )
    return _gather_concat(indices, *tables)
